# baseline (speedup 1.0000x reference)
import functools

import jax
import jax.numpy as jnp
from jax import lax
from jax.experimental import pallas as pl
from jax.experimental.pallas import tpu as pltpu
from jax.experimental.pallas import tpu_sc as plsc

N = 10000
E = 320000
D = 128
H = 512

SC_CORES = 2
SC_SUBCORES = 16
NW = SC_CORES * SC_SUBCORES
NP = 10240
KI = 128
NCH = E // KI
NCH2 = 2504
CH_W = NCH // NW
CH_X = NCH - CH_W * NW
CH_A = 40
CH_B = CH_W - CH_A
CH_STG = 48
CH_DEG = NCH // SC_SUBCORES
DEG_X = NCH - CH_DEG * SC_SUBCORES
DEG_STG = 168
RPT = NP // SC_SUBCORES

BR = 1024
GRID = (N + BR - 1) // BR


def _deg_body(src_hbm, dst_hbm, zeros_hbm, ones_hbm, out_hbm,
              idx_v, ones_v, deg_sp):
    c = lax.axis_index("c")
    s = lax.axis_index("s")
    pltpu.sync_copy(zeros_hbm.at[pl.ds(s * RPT, RPT)],
                    deg_sp.at[pl.ds(s * RPT, RPT)])
    pltpu.sync_copy(ones_hbm, ones_v)
    start = s * CH_DEG
    stg = pl.multiple_of(jnp.minimum((start // 8) * 8, NCH2 - DEG_STG), 8)
    dlt = start - stg

    @pl.when(c == 0)
    def _():
        pltpu.sync_copy(src_hbm.at[pl.ds(stg, DEG_STG)], idx_v)

    @pl.when(c == 1)
    def _():
        pltpu.sync_copy(dst_hbm.at[pl.ds(stg, DEG_STG)], idx_v)

    plsc.subcore_barrier()

    def body(j, carry):
        pltpu.sync_copy(ones_v, deg_sp.at[idx_v.at[dlt + j]], add=True)
        return carry

    lax.fori_loop(0, CH_DEG, body, 0)

    @pl.when(s < DEG_X)
    def _():
        tail = CH_DEG * SC_SUBCORES - (NCH2 - 8)

        @pl.when(c == 0)
        def _():
            pltpu.sync_copy(src_hbm.at[pl.ds(NCH2 - 8, 8)],
                            idx_v.at[pl.ds(0, 8)])

        @pl.when(c == 1)
        def _():
            pltpu.sync_copy(dst_hbm.at[pl.ds(NCH2 - 8, 8)],
                            idx_v.at[pl.ds(0, 8)])

        pltpu.sync_copy(ones_v, deg_sp.at[idx_v.at[tail + s]], add=True)

    plsc.subcore_barrier()
    pltpu.sync_copy(deg_sp.at[pl.ds(s * RPT, RPT)],
                    out_hbm.at[c, pl.ds(s * RPT, RPT)])


@functools.cache
def _sc_mesh():
    return plsc.VectorSubcoreMesh(
        core_axis_name="c", subcore_axis_name="s",
        num_cores=SC_CORES, num_subcores=SC_SUBCORES)


@functools.cache
def _deg_kernel_build():
    return pl.kernel(
        _deg_body,
        out_type=jax.ShapeDtypeStruct((2, NP), jnp.float32),
        mesh=_sc_mesh(),
        scratch_types=[
            pltpu.VMEM((DEG_STG, KI), jnp.int32),
            pltpu.VMEM((KI,), jnp.float32),
            pltpu.VMEM_SHARED((NP,), jnp.float32),
        ],
    )


def _deg_kernel(*args):
    return _deg_kernel_build()(*args)


def _spmm_body(p_hbm, src_hbm, dst_hbm, zeros_hbm, out_hbm,
               sidx_v, didx_v, rows0_v, rows1_v, agg_sp,
               sg0, sg1, ss0, ss1):
    c = lax.axis_index("c")
    s = lax.axis_index("s")
    pltpu.sync_copy(zeros_hbm.at[pl.ds(s * RPT, RPT)],
                    agg_sp.at[pl.ds(s * RPT, RPT)])
    wid = c * SC_SUBCORES + s
    base = wid * CH_W

    def drain(rows_v, sem):
        pltpu.make_async_copy(p_hbm.at[pl.ds(0, KI)], rows_v, sem).wait()

    def half(start, nch, first):
        stg = pl.multiple_of(jnp.minimum((start // 8) * 8, NCH2 - CH_STG), 8)
        dlt = start - stg
        pltpu.sync_copy(src_hbm.at[pl.ds(stg, CH_STG)], sidx_v)
        pltpu.sync_copy(dst_hbm.at[pl.ds(stg, CH_STG)], didx_v)
        if first:
            plsc.subcore_barrier()
        pltpu.async_copy(p_hbm.at[sidx_v.at[dlt]], rows0_v, sg0)
        pltpu.async_copy(p_hbm.at[sidx_v.at[dlt + 1]], rows1_v, sg1)

        def body(g, carry):
            drain(rows0_v, sg0)
            pltpu.async_copy(rows0_v, agg_sp.at[didx_v.at[dlt + 2 * g]],
                             ss0, add=True)
            drain(rows1_v, sg1)
            pltpu.async_copy(rows1_v, agg_sp.at[didx_v.at[dlt + 2 * g + 1]],
                             ss1, add=True)
            nxt0 = jnp.minimum(2 * g + 2, nch - 1)
            drain(rows0_v, ss0)
            pltpu.async_copy(p_hbm.at[sidx_v.at[dlt + nxt0]], rows0_v, sg0)
            nxt1 = jnp.minimum(2 * g + 3, nch - 1)
            drain(rows1_v, ss1)
            pltpu.async_copy(p_hbm.at[sidx_v.at[dlt + nxt1]], rows1_v, sg1)
            return carry

        lax.fori_loop(0, nch // 2, body, 0)
        drain(rows0_v, sg0)
        drain(rows1_v, sg1)

    half(base, CH_A, True)
    half(base + CH_A, CH_B, False)

    @pl.when(wid < CH_X)
    def _():
        tail = CH_W * NW - (NCH2 - 8)
        pltpu.sync_copy(src_hbm.at[pl.ds(NCH2 - 8, 8)], sidx_v.at[pl.ds(0, 8)])
        pltpu.sync_copy(dst_hbm.at[pl.ds(NCH2 - 8, 8)], didx_v.at[pl.ds(0, 8)])
        pltpu.sync_copy(p_hbm.at[sidx_v.at[tail + wid]], rows0_v)
        pltpu.sync_copy(rows0_v, agg_sp.at[didx_v.at[tail + wid]], add=True)

    plsc.subcore_barrier()
    pltpu.sync_copy(agg_sp.at[pl.ds(s * RPT, RPT)],
                    out_hbm.at[c, pl.ds(s * RPT, RPT)])


@functools.cache
def _spmm_kernel_build():
    return pl.kernel(
        _spmm_body,
        out_type=jax.ShapeDtypeStruct((2, NP, D), jnp.float32),
        mesh=_sc_mesh(),
        scratch_types=[
            pltpu.VMEM((CH_STG, KI), jnp.int32),
            pltpu.VMEM((CH_STG, KI), jnp.int32),
            pltpu.VMEM((KI, D), jnp.float32),
            pltpu.VMEM((KI, D), jnp.float32),
            pltpu.VMEM_SHARED((NP, D), jnp.float32),
            pltpu.SemaphoreType.DMA,
            pltpu.SemaphoreType.DMA,
            pltpu.SemaphoreType.DMA,
            pltpu.SemaphoreType.DMA,
        ],
    )


def _spmm_kernel(*args):
    return _spmm_kernel_build()(*args)


def _mlp_block(feat_ref, lab_ref, tab_ref, w1a_ref, w1b_ref, b1_ref,
               g_ref, bln_ref, w2_ref, b2_ref, deg_ref, out_ref):
    lab = lab_ref[...]
    oh = (lax.broadcasted_iota(jnp.int32, (BR, 128), 1) == lab)
    emb = jnp.dot(oh.astype(jnp.float32), tab_ref[...],
                  preferred_element_type=jnp.float32)
    h = (jnp.dot(feat_ref[...], w1a_ref[...], preferred_element_type=jnp.float32)
         + jnp.dot(emb, w1b_ref[...], preferred_element_type=jnp.float32)
         + b1_ref[...])
    mu = jnp.mean(h, axis=-1, keepdims=True)
    hc = h - mu
    var = jnp.mean(hc * hc, axis=-1, keepdims=True)
    h = hc * lax.rsqrt(var + 1e-5) * g_ref[...] + bln_ref[...]
    h = jnp.maximum(h, 0.0)
    h = jnp.dot(h, w2_ref[...], preferred_element_type=jnp.float32) + b2_ref[...]
    nsrc = lax.rsqrt(jnp.maximum(deg_ref[...][:, 0:1], 1.0))
    out_ref[...] = h * nsrc


_mlp_call = pl.pallas_call(
    _mlp_block,
    grid=(GRID,),
    in_specs=[
        pl.BlockSpec((BR, D), lambda i: (i, 0)),
        pl.BlockSpec((BR, 1), lambda i: (i, 0)),
        pl.BlockSpec((128, D), lambda i: (0, 0)),
        pl.BlockSpec((D, H), lambda i: (0, 0)),
        pl.BlockSpec((D, H), lambda i: (0, 0)),
        pl.BlockSpec((1, H), lambda i: (0, 0)),
        pl.BlockSpec((1, H), lambda i: (0, 0)),
        pl.BlockSpec((1, H), lambda i: (0, 0)),
        pl.BlockSpec((H, D), lambda i: (0, 0)),
        pl.BlockSpec((1, D), lambda i: (0, 0)),
        pl.BlockSpec((BR, 2), lambda i: (i, 0)),
    ],
    out_specs=pl.BlockSpec((BR, D), lambda i: (i, 0)),
    out_shape=jax.ShapeDtypeStruct((N, D), jnp.float32),
)


def _conv_mid_block(agg_ref, deg_ref, w_ref, b_ref, pn_ref):
    a = agg_ref[0] + agg_ref[1]
    dd = deg_ref[...]
    ndst = lax.rsqrt(jnp.maximum(dd[:, 1:2], 1.0))
    nsrc = lax.rsqrt(jnp.maximum(dd[:, 0:1], 1.0))
    h = jnp.dot(a * ndst, w_ref[...], preferred_element_type=jnp.float32)
    pn_ref[...] = jnp.maximum(h + b_ref[...], 0.0) * nsrc


def _conv_out_block(agg_ref, deg_ref, w_ref, b_ref, h_ref):
    a = agg_ref[0] + agg_ref[1]
    ndst = lax.rsqrt(jnp.maximum(deg_ref[...][:, 1:2], 1.0))
    h = jnp.dot(a * ndst, w_ref[...], preferred_element_type=jnp.float32)
    h_ref[...] = jnp.maximum(h + b_ref[...], 0.0)


def _conv_call(body):
    return pl.pallas_call(
        body,
        grid=(GRID,),
        in_specs=[
            pl.BlockSpec((2, BR, D), lambda i: (0, i, 0)),
            pl.BlockSpec((BR, 2), lambda i: (i, 0)),
            pl.BlockSpec((D, D), lambda i: (0, 0)),
            pl.BlockSpec((1, D), lambda i: (0, 0)),
        ],
        out_specs=pl.BlockSpec((BR, D), lambda i: (i, 0)),
        out_shape=jax.ShapeDtypeStruct((N, D), jnp.float32),
    )


_conv_mid = _conv_call(_conv_mid_block)
_conv_out = _conv_call(_conv_out_block)


def kernel(features, edge_index, input_labels, label_table,
           W1, b1, ln_g, ln_b, W2, b2, Wc0, bc0, Wc1, bc1):
    f32 = jnp.float32
    labc = input_labels[:, None]
    tabp = jnp.pad(label_table, ((0, 128 - label_table.shape[0]), (0, 0)))
    w1a, w1b = W1[:D], W1[D:]
    b1r, gr, blnr = b1[None, :], ln_g[None, :], ln_b[None, :]
    b2r, bc0r, bc1r = b2[None, :], bc0[None, :], bc1[None, :]

    src2d = jnp.pad(edge_index[0].reshape(NCH, KI), ((0, NCH2 - NCH), (0, 0)))
    dst2d = jnp.pad(edge_index[1].reshape(NCH, KI), ((0, NCH2 - NCH), (0, 0)))

    zeros_nd = jnp.zeros((NP, D), f32)
    zeros_n = jnp.zeros((NP,), f32)
    ones_k = jnp.ones((KI,), f32)

    deg2 = _deg_kernel(src2d, dst2d, zeros_n, ones_k)
    deg = deg2.T

    p = _mlp_call(features, labc, tabp, w1a, w1b, b1r, gr, blnr, W2, b2r, deg)

    agg = _spmm_kernel(p, src2d, dst2d, zeros_nd)
    p = _conv_mid(agg, deg, Wc0, bc0r)
    agg = _spmm_kernel(p, src2d, dst2d, zeros_nd)
    return _conv_out(agg, deg, Wc1, bc1r)


# speedup vs baseline: 1.2347x; 1.2347x over previous
import functools

import jax
import jax.numpy as jnp
from jax import lax
from jax.experimental import pallas as pl
from jax.experimental.pallas import tpu as pltpu
from jax.experimental.pallas import tpu_sc as plsc

N = 10000
E = 320000
D = 128
H = 512

SC_CORES = 2
SC_SUBCORES = 16
NW = SC_CORES * SC_SUBCORES
NP = 10240
KI = 128
NCH = E // KI
NCH2 = 2504
CH_W = NCH // NW
CH_X = NCH - CH_W * NW
CH_A = 40
CH_B = CH_W - CH_A
CH_STG = 48
CH_DEG = NCH // SC_SUBCORES
DEG_X = NCH - CH_DEG * SC_SUBCORES
DEG_STG = 168
RPT = NP // SC_SUBCORES

BR = 1024
GRID = (N + BR - 1) // BR


def _deg_body(src_hbm, dst_hbm, zeros_hbm, ones_hbm, out_hbm,
              idx_v, ones_v, deg_sp):
    c = lax.axis_index("c")
    s = lax.axis_index("s")
    pltpu.sync_copy(zeros_hbm.at[pl.ds(s * RPT, RPT)],
                    deg_sp.at[pl.ds(s * RPT, RPT)])
    pltpu.sync_copy(ones_hbm, ones_v)
    start = s * CH_DEG
    stg = pl.multiple_of(jnp.minimum((start // 8) * 8, NCH2 - DEG_STG), 8)
    dlt = start - stg

    @pl.when(c == 0)
    def _():
        pltpu.sync_copy(src_hbm.at[pl.ds(stg, DEG_STG)], idx_v)

    @pl.when(c == 1)
    def _():
        pltpu.sync_copy(dst_hbm.at[pl.ds(stg, DEG_STG)], idx_v)

    plsc.subcore_barrier()

    def body(j, carry):
        pltpu.sync_copy(ones_v, deg_sp.at[idx_v.at[dlt + j]], add=True)
        return carry

    lax.fori_loop(0, CH_DEG, body, 0)

    @pl.when(s < DEG_X)
    def _():
        tail = CH_DEG * SC_SUBCORES - (NCH2 - 8)

        @pl.when(c == 0)
        def _():
            pltpu.sync_copy(src_hbm.at[pl.ds(NCH2 - 8, 8)],
                            idx_v.at[pl.ds(0, 8)])

        @pl.when(c == 1)
        def _():
            pltpu.sync_copy(dst_hbm.at[pl.ds(NCH2 - 8, 8)],
                            idx_v.at[pl.ds(0, 8)])

        pltpu.sync_copy(ones_v, deg_sp.at[idx_v.at[tail + s]], add=True)

    plsc.subcore_barrier()
    pltpu.sync_copy(deg_sp.at[pl.ds(s * RPT, RPT)],
                    out_hbm.at[c, pl.ds(s * RPT, RPT)])


@functools.cache
def _sc_mesh():
    return plsc.VectorSubcoreMesh(
        core_axis_name="c", subcore_axis_name="s",
        num_cores=SC_CORES, num_subcores=SC_SUBCORES)


@functools.cache
def _deg_kernel_build():
    return pl.kernel(
        _deg_body,
        out_type=jax.ShapeDtypeStruct((2, NP), jnp.float32),
        mesh=_sc_mesh(),
        scratch_types=[
            pltpu.VMEM((DEG_STG, KI), jnp.int32),
            pltpu.VMEM((KI,), jnp.float32),
            pltpu.VMEM_SHARED((NP,), jnp.float32),
        ],
    )


def _deg_kernel(*args):
    return _deg_kernel_build()(*args)


def _spmm_body(p_hbm, src_hbm, dst_hbm, zeros_hbm, out_hbm,
               sidx_v, didx_v, rows0_v, rows1_v, agg_sp, sem0, sem1):
    c = lax.axis_index("c")
    s = lax.axis_index("s")
    pltpu.sync_copy(zeros_hbm.at[pl.ds(s * RPT, RPT)],
                    agg_sp.at[pl.ds(s * RPT, RPT)])
    wid = c * SC_SUBCORES + s
    base = wid * CH_W

    def drain(rows_v, sem):
        pltpu.make_async_copy(p_hbm.at[pl.ds(0, KI)], rows_v, sem).wait()

    def half(start, nch, first):
        stg = pl.multiple_of(jnp.minimum((start // 8) * 8, NCH2 - CH_STG), 8)
        dlt = start - stg
        pltpu.sync_copy(src_hbm.at[pl.ds(stg, CH_STG)], sidx_v)
        pltpu.sync_copy(dst_hbm.at[pl.ds(stg, CH_STG)], didx_v)
        if first:
            plsc.subcore_barrier()
        pltpu.async_copy(p_hbm.at[sidx_v.at[dlt]], rows0_v, sem0)
        pltpu.async_copy(p_hbm.at[sidx_v.at[dlt + 1]], rows1_v, sem1)

        def body(g, carry):
            drain(rows0_v, sem0)
            pltpu.sync_copy(rows0_v, agg_sp.at[didx_v.at[dlt + 2 * g]],
                            add=True)
            nxt0 = jnp.minimum(2 * g + 2, nch - 1)
            pltpu.async_copy(p_hbm.at[sidx_v.at[dlt + nxt0]], rows0_v, sem0)
            drain(rows1_v, sem1)
            pltpu.sync_copy(rows1_v, agg_sp.at[didx_v.at[dlt + 2 * g + 1]],
                            add=True)
            nxt1 = jnp.minimum(2 * g + 3, nch - 1)
            pltpu.async_copy(p_hbm.at[sidx_v.at[dlt + nxt1]], rows1_v, sem1)
            return carry

        lax.fori_loop(0, nch // 2, body, 0)
        drain(rows0_v, sem0)
        drain(rows1_v, sem1)

    half(base, CH_A, True)
    half(base + CH_A, CH_B, False)

    @pl.when(wid < CH_X)
    def _():
        tail = CH_W * NW - (NCH2 - 8)
        pltpu.sync_copy(src_hbm.at[pl.ds(NCH2 - 8, 8)], sidx_v.at[pl.ds(0, 8)])
        pltpu.sync_copy(dst_hbm.at[pl.ds(NCH2 - 8, 8)], didx_v.at[pl.ds(0, 8)])
        pltpu.sync_copy(p_hbm.at[sidx_v.at[tail + wid]], rows0_v)
        pltpu.sync_copy(rows0_v, agg_sp.at[didx_v.at[tail + wid]], add=True)

    plsc.subcore_barrier()
    pltpu.sync_copy(agg_sp.at[pl.ds(s * RPT, RPT)],
                    out_hbm.at[c, pl.ds(s * RPT, RPT)])


@functools.cache
def _spmm_kernel_build():
    return pl.kernel(
        _spmm_body,
        out_type=jax.ShapeDtypeStruct((2, NP, D), jnp.float32),
        mesh=_sc_mesh(),
        scratch_types=[
            pltpu.VMEM((CH_STG, KI), jnp.int32),
            pltpu.VMEM((CH_STG, KI), jnp.int32),
            pltpu.VMEM((KI, D), jnp.float32),
            pltpu.VMEM((KI, D), jnp.float32),
            pltpu.VMEM_SHARED((NP, D), jnp.float32),
            pltpu.SemaphoreType.DMA,
            pltpu.SemaphoreType.DMA,
        ],
    )


def _spmm_kernel(*args):
    return _spmm_kernel_build()(*args)


def _mlp_block(feat_ref, lab_ref, tab_ref, w1a_ref, w1b_ref, b1_ref,
               g_ref, bln_ref, w2_ref, b2_ref, out_ref):
    lab = lab_ref[...]
    oh = (lax.broadcasted_iota(jnp.int32, (BR, 128), 1) == lab)
    emb = jnp.dot(oh.astype(jnp.float32), tab_ref[...],
                  preferred_element_type=jnp.float32)
    h = (jnp.dot(feat_ref[...], w1a_ref[...], preferred_element_type=jnp.float32)
         + jnp.dot(emb, w1b_ref[...], preferred_element_type=jnp.float32)
         + b1_ref[...])
    mu = jnp.mean(h, axis=-1, keepdims=True)
    hc = h - mu
    var = jnp.mean(hc * hc, axis=-1, keepdims=True)
    h = hc * lax.rsqrt(var + 1e-5) * g_ref[...] + bln_ref[...]
    h = jnp.maximum(h, 0.0)
    out_ref[...] = (jnp.dot(h, w2_ref[...], preferred_element_type=jnp.float32)
                    + b2_ref[...])


_mlp_call = pl.pallas_call(
    _mlp_block,
    grid=(GRID,),
    in_specs=[
        pl.BlockSpec((BR, D), lambda i: (i, 0)),
        pl.BlockSpec((BR, 1), lambda i: (i, 0)),
        pl.BlockSpec((128, D), lambda i: (0, 0)),
        pl.BlockSpec((D, H), lambda i: (0, 0)),
        pl.BlockSpec((D, H), lambda i: (0, 0)),
        pl.BlockSpec((1, H), lambda i: (0, 0)),
        pl.BlockSpec((1, H), lambda i: (0, 0)),
        pl.BlockSpec((1, H), lambda i: (0, 0)),
        pl.BlockSpec((H, D), lambda i: (0, 0)),
        pl.BlockSpec((1, D), lambda i: (0, 0)),
    ],
    out_specs=pl.BlockSpec((BR, D), lambda i: (i, 0)),
    out_shape=jax.ShapeDtypeStruct((N, D), jnp.float32),
)


def _scale_block(h_ref, deg_ref, out_ref):
    nsrc = lax.rsqrt(jnp.maximum(deg_ref[...][:, 0:1], 1.0))
    out_ref[...] = h_ref[...] * nsrc


_scale_call = pl.pallas_call(
    _scale_block,
    grid=(GRID,),
    in_specs=[
        pl.BlockSpec((BR, D), lambda i: (i, 0)),
        pl.BlockSpec((BR, 2), lambda i: (i, 0)),
    ],
    out_specs=pl.BlockSpec((BR, D), lambda i: (i, 0)),
    out_shape=jax.ShapeDtypeStruct((N, D), jnp.float32),
)


def _conv_mid_block(agg_ref, deg_ref, w_ref, b_ref, pn_ref):
    a = agg_ref[0] + agg_ref[1]
    dd = deg_ref[...]
    ndst = lax.rsqrt(jnp.maximum(dd[:, 1:2], 1.0))
    nsrc = lax.rsqrt(jnp.maximum(dd[:, 0:1], 1.0))
    h = jnp.dot(a * ndst, w_ref[...], preferred_element_type=jnp.float32)
    pn_ref[...] = jnp.maximum(h + b_ref[...], 0.0) * nsrc


def _conv_out_block(agg_ref, deg_ref, w_ref, b_ref, h_ref):
    a = agg_ref[0] + agg_ref[1]
    ndst = lax.rsqrt(jnp.maximum(deg_ref[...][:, 1:2], 1.0))
    h = jnp.dot(a * ndst, w_ref[...], preferred_element_type=jnp.float32)
    h_ref[...] = jnp.maximum(h + b_ref[...], 0.0)


def _conv_call(body):
    return pl.pallas_call(
        body,
        grid=(GRID,),
        in_specs=[
            pl.BlockSpec((2, BR, D), lambda i: (0, i, 0)),
            pl.BlockSpec((BR, 2), lambda i: (i, 0)),
            pl.BlockSpec((D, D), lambda i: (0, 0)),
            pl.BlockSpec((1, D), lambda i: (0, 0)),
        ],
        out_specs=pl.BlockSpec((BR, D), lambda i: (i, 0)),
        out_shape=jax.ShapeDtypeStruct((N, D), jnp.float32),
    )


_conv_mid = _conv_call(_conv_mid_block)
_conv_out = _conv_call(_conv_out_block)


def kernel(features, edge_index, input_labels, label_table,
           W1, b1, ln_g, ln_b, W2, b2, Wc0, bc0, Wc1, bc1):
    f32 = jnp.float32
    labc = input_labels[:, None]
    tabp = jnp.pad(label_table, ((0, 128 - label_table.shape[0]), (0, 0)))
    w1a, w1b = W1[:D], W1[D:]
    b1r, gr, blnr = b1[None, :], ln_g[None, :], ln_b[None, :]
    b2r, bc0r, bc1r = b2[None, :], bc0[None, :], bc1[None, :]

    src2d = jnp.pad(edge_index[0].reshape(NCH, KI), ((0, NCH2 - NCH), (0, 0)))
    dst2d = jnp.pad(edge_index[1].reshape(NCH, KI), ((0, NCH2 - NCH), (0, 0)))

    zeros_nd = jnp.zeros((NP, D), f32)
    zeros_n = jnp.zeros((NP,), f32)
    ones_k = jnp.ones((KI,), f32)

    deg2 = _deg_kernel(src2d, dst2d, zeros_n, ones_k)
    deg = deg2.T

    h0 = _mlp_call(features, labc, tabp, w1a, w1b, b1r, gr, blnr, W2, b2r)
    p = _scale_call(h0, deg)

    agg = _spmm_kernel(p, src2d, dst2d, zeros_nd)
    p = _conv_mid(agg, deg, Wc0, bc0r)
    agg = _spmm_kernel(p, src2d, dst2d, zeros_nd)
    return _conv_out(agg, deg, Wc1, bc1r)
